# Initial kernel scaffold; baseline (speedup 1.0000x reference)
import jax
import jax.numpy as jnp
from jax.experimental import pallas as pl


def kernel(q, k, v, null_keys, null_values, sort_linear):
    raise NotImplementedError("write your pallas kernel here")


# speedup vs baseline: 10.0974x; 10.0974x over previous
import functools

import jax
import jax.numpy as jnp
from jax.experimental import pallas as pl
from jax.experimental.pallas import tpu as pltpu

_BUCKETS = 64
_DIM = 1024


def _fused_body(q_ref, k_ref, v_ref, w_ref, nk_ref, nv_ref, o_ref,
                kvk_ref, kvv_ref, bkr_ref, bvr_ref, rsm_ref, *,
                h, hh, t, dh, buckets, bsz):
    neg = -jnp.finfo(jnp.float32).max
    pid = pl.program_id(0)
    is_rolled = (pid % h) >= hh

    shift = bsz - 1

    def roll_fwd(x):
        return jnp.concatenate([x[shift:], x[:shift]], axis=0)

    q = q_ref[0]
    k = k_ref[0]
    v = v_ref[0]
    q = jnp.where(is_rolled, roll_fwd(q), q)
    k = jnp.where(is_rolled, roll_fwd(k), k)
    v = jnp.where(is_rolled, roll_fwd(v), v)

    kb = k.reshape(buckets, bsz, dh)
    vb = v.reshape(buckets, bsz, dh)
    qb = q.reshape(buckets, bsz, dh)

    bsums = jnp.sum(kb, axis=1)
    tri = (jax.lax.broadcasted_iota(jnp.int32, (buckets, buckets), 0)
           > jax.lax.broadcasted_iota(jnp.int32, (buckets, buckets), 1)
           ).astype(jnp.float32)
    excl = jnp.dot(tri, bsums, preferred_element_type=jnp.float32)
    firsts = kb[:, 0, :]
    pos = (jax.lax.broadcasted_iota(jnp.int32, (buckets, 1), 0) * bsz + 1
           ).astype(jnp.float32)
    x1 = (excl + firsts) / pos
    x = jnp.concatenate([x1, firsts], axis=1)

    r_raw = jnp.dot(x, w_ref[0], preferred_element_type=jnp.float32)
    r_act = jnp.where(r_raw >= 0, r_raw, 0.01 * r_raw)
    rows = jax.lax.broadcasted_iota(jnp.int32, (buckets, buckets + 1), 0)
    cols = jax.lax.broadcasted_iota(jnp.int32, (buckets, buckets + 1), 1)
    r_m = jnp.where(cols > rows, neg, r_act)
    r_m = r_m - jnp.max(r_m, axis=1, keepdims=True)
    r_e = jnp.exp(r_m)
    r_sm = r_e / jnp.sum(r_e, axis=1, keepdims=True)
    r_sm = jnp.where(cols <= rows - 1, r_sm, 0.0)
    rsm_ref[...] = r_sm

    kvk_ref[0:bsz, :] = jnp.broadcast_to(nk_ref[0], (bsz, dh))
    kvv_ref[0:bsz, :] = jnp.broadcast_to(nv_ref[0], (bsz, dh))
    kvk_ref[bsz:, :] = k
    kvv_ref[bsz:, :] = v
    cols_row = jax.lax.broadcasted_iota(jnp.int32, (1, buckets + 1), 1)

    def gather_one(u, carry):
        row = rsm_ref[pl.ds(u, 1), :]
        w_u = jnp.max(row)
        ti = jnp.min(jnp.where(row == w_u, cols_row, buckets + 1))
        src = ti * bsz
        bkr_ref[pl.ds(u * bsz, bsz), :] = w_u * kvk_ref[pl.ds(src, bsz), :]
        bvr_ref[pl.ds(u * bsz, bsz), :] = w_u * kvv_ref[pl.ds(src, bsz), :]
        return carry

    jax.lax.fori_loop(0, buckets, gather_one, 0, unroll=True)
    bkr = bkr_ref[...].reshape(buckets, bsz, dh)
    bvr = bvr_ref[...].reshape(buckets, bsz, dh)

    k2 = jnp.concatenate([bkr, kb], axis=1)
    v2 = jnp.concatenate([bvr, vb], axis=1)
    dots = jax.lax.dot_general(
        qb, k2, (((2,), (2,)), ((0,), (0,))),
        preferred_element_type=jnp.float32) * (float(_DIM) ** -0.5)

    mshape = (buckets, bsz, 2 * bsz)
    ii = jax.lax.broadcasted_iota(jnp.int32, mshape, 1)
    jj = jax.lax.broadcasted_iota(jnp.int32, mshape, 2)
    ub = jax.lax.broadcasted_iota(jnp.int32, mshape, 0)
    base = ~((jj >= bsz) & ((jj - bsz) > ii))
    early = jj < bsz + 1
    special = ((ii == 0) & early) | (base & ~early)
    use_special = jnp.logical_and(is_rolled, ub == buckets - 1)
    mask = (use_special & special) | (~use_special & base)

    dots = jnp.where(mask, dots, neg)
    dots = dots - jnp.max(dots, axis=2, keepdims=True)
    e = jnp.exp(dots)
    attn = e / jnp.sum(e, axis=2, keepdims=True)
    ob = jax.lax.dot_general(
        attn, v2, (((2,), (1,)), ((0,), (0,))),
        preferred_element_type=jnp.float32)

    o = ob.reshape(t, dh)
    o_roll = jnp.concatenate([o[t - shift:], o[:t - shift]], axis=0)
    o_ref[0] = jnp.where(is_rolled, o_roll, o)


def kernel(q, k, v, null_keys, null_values, sort_linear):
    b, h, t, dh = q.shape
    bh = b * h
    buckets = _BUCKETS
    bsz = t // buckets
    hh = h // 2

    qf = q.reshape(bh, t, dh)
    kf = k.reshape(bh, t, dh)
    vf = v.reshape(bh, t, dh)
    w = sort_linear.reshape(h, 2 * dh, buckets + 1)
    nk = null_keys.reshape(h, 1, dh)
    nv = null_values.reshape(h, 1, dh)

    body = functools.partial(_fused_body, h=h, hh=hh, t=t, dh=dh,
                             buckets=buckets, bsz=bsz)
    out = pl.pallas_call(
        body,
        grid=(bh,),
        in_specs=[
            pl.BlockSpec((1, t, dh), lambda i: (i, 0, 0)),
            pl.BlockSpec((1, t, dh), lambda i: (i, 0, 0)),
            pl.BlockSpec((1, t, dh), lambda i: (i, 0, 0)),
            pl.BlockSpec((1, 2 * dh, buckets + 1), lambda i, h=h: (i % h, 0, 0)),
            pl.BlockSpec((1, 1, dh), lambda i, h=h: (i % h, 0, 0)),
            pl.BlockSpec((1, 1, dh), lambda i, h=h: (i % h, 0, 0)),
        ],
        out_specs=pl.BlockSpec((1, t, dh), lambda i: (i, 0, 0)),
        out_shape=jax.ShapeDtypeStruct((bh, t, dh), jnp.float32),
        scratch_shapes=[
            pltpu.VMEM((bsz + t, dh), jnp.float32),
            pltpu.VMEM((bsz + t, dh), jnp.float32),
            pltpu.VMEM((t, dh), jnp.float32),
            pltpu.VMEM((t, dh), jnp.float32),
            pltpu.VMEM((buckets, buckets + 1), jnp.float32),
        ],
        compiler_params=pltpu.CompilerParams(
            dimension_semantics=("arbitrary",)),
    )(qf, kf, vf, w, nk, nv)
    return out.reshape(b, h, t, dh)


# speedup vs baseline: 21.4512x; 2.1244x over previous
import functools

import jax
import jax.numpy as jnp
from jax.experimental import pallas as pl
from jax.experimental.pallas import tpu as pltpu

_BUCKETS = 64
_DIM = 1024


def _fused_body(q_ref, k_ref, v_ref, w_ref, nk_ref, nv_ref, o_ref, *,
                h, hh, t, dh, buckets, bsz):
    neg = -jnp.finfo(jnp.float32).max
    pid = pl.program_id(0)
    is_rolled = (pid % h) >= hh

    shift = bsz - 1

    def roll_fwd(x):
        return jnp.concatenate([x[shift:], x[:shift]], axis=0)

    q = q_ref[0]
    k = k_ref[0]
    v = v_ref[0]
    q = jnp.where(is_rolled, roll_fwd(q), q)
    k = jnp.where(is_rolled, roll_fwd(k), k)
    v = jnp.where(is_rolled, roll_fwd(v), v)

    kb = k.reshape(buckets, bsz, dh)
    vb = v.reshape(buckets, bsz, dh)
    qb = q.reshape(buckets, bsz, dh)

    bsums = jnp.sum(kb, axis=1)
    tri = (jax.lax.broadcasted_iota(jnp.int32, (buckets, buckets), 0)
           > jax.lax.broadcasted_iota(jnp.int32, (buckets, buckets), 1)
           ).astype(jnp.float32)
    excl = jnp.dot(tri, bsums, preferred_element_type=jnp.float32)
    firsts = kb[:, 0, :]
    pos = (jax.lax.broadcasted_iota(jnp.int32, (buckets, 1), 0) * bsz + 1
           ).astype(jnp.float32)
    x1 = (excl + firsts) / pos
    x = jnp.concatenate([x1, firsts], axis=1)

    r_raw = jnp.dot(x, w_ref[0], preferred_element_type=jnp.float32)
    r_act = jnp.where(r_raw >= 0, r_raw, 0.01 * r_raw)
    rows = jax.lax.broadcasted_iota(jnp.int32, (buckets, buckets + 1), 0)
    cols = jax.lax.broadcasted_iota(jnp.int32, (buckets, buckets + 1), 1)
    r_m = jnp.where(cols > rows, neg, r_act)
    r_m = r_m - jnp.max(r_m, axis=1, keepdims=True)
    r_e = jnp.exp(r_m)
    r_sm = r_e / jnp.sum(r_e, axis=1, keepdims=True)
    r_sm = jnp.where(cols <= rows - 1, r_sm, 0.0)

    mx_v = jnp.max(r_sm, axis=1, keepdims=True)
    top_v = jnp.min(jnp.where(r_sm == mx_v, cols, buckets + 1), axis=1,
                    keepdims=True)
    r_kept = jnp.where(cols == top_v, r_sm, 0.0)
    nk_tile = jnp.broadcast_to(nk_ref[0], (bsz, dh))
    nv_tile = jnp.broadcast_to(nv_ref[0], (bsz, dh))
    kv_ext_k = jnp.concatenate([nk_tile[None], kb], axis=0)
    kv_ext_v = jnp.concatenate([nv_tile[None], vb], axis=0)
    bkr = jax.lax.dot_general(
        r_kept, kv_ext_k, (((1,), (0,)), ((), ())),
        preferred_element_type=jnp.float32)
    bvr = jax.lax.dot_general(
        r_kept, kv_ext_v, (((1,), (0,)), ((), ())),
        preferred_element_type=jnp.float32)

    k2 = jnp.concatenate([bkr, kb], axis=1)
    v2 = jnp.concatenate([bvr, vb], axis=1)
    dots = jax.lax.dot_general(
        qb, k2, (((2,), (2,)), ((0,), (0,))),
        preferred_element_type=jnp.float32) * (float(_DIM) ** -0.5)

    ii2 = jax.lax.broadcasted_iota(jnp.int32, (bsz, 2 * bsz), 0)
    jj2 = jax.lax.broadcasted_iota(jnp.int32, (bsz, 2 * bsz), 1)
    base2 = ~((jj2 >= bsz) & ((jj2 - bsz) > ii2))
    early2 = jj2 < bsz + 1
    special2 = ((ii2 == 0) & early2) | (base2 & ~early2)
    base_f = jnp.where(base2, 0.0, neg)
    special_f = jnp.where(special2, 0.0, neg)
    last_f = jnp.where(is_rolled, special_f, base_f)
    ub = jax.lax.broadcasted_iota(jnp.int32, (buckets, 1, 1), 0)
    mask3 = jnp.where(ub == buckets - 1, last_f[None], base_f[None])
    dots = dots + mask3
    dots = dots - jnp.max(dots, axis=2, keepdims=True)
    e = jnp.exp(dots)
    attn = e / jnp.sum(e, axis=2, keepdims=True)
    ob = jax.lax.dot_general(
        attn, v2, (((2,), (1,)), ((0,), (0,))),
        preferred_element_type=jnp.float32)

    o = ob.reshape(t, dh)
    o_roll = jnp.concatenate([o[t - shift:], o[:t - shift]], axis=0)
    o_ref[0] = jnp.where(is_rolled, o_roll, o)


def kernel(q, k, v, null_keys, null_values, sort_linear):
    b, h, t, dh = q.shape
    bh = b * h
    buckets = _BUCKETS
    bsz = t // buckets
    hh = h // 2

    qf = q.reshape(bh, t, dh)
    kf = k.reshape(bh, t, dh)
    vf = v.reshape(bh, t, dh)
    w = sort_linear.reshape(h, 2 * dh, buckets + 1)
    nk = null_keys.reshape(h, 1, dh)
    nv = null_values.reshape(h, 1, dh)

    body = functools.partial(_fused_body, h=h, hh=hh, t=t, dh=dh,
                             buckets=buckets, bsz=bsz)
    out = pl.pallas_call(
        body,
        grid=(bh,),
        in_specs=[
            pl.BlockSpec((1, t, dh), lambda i: (i, 0, 0)),
            pl.BlockSpec((1, t, dh), lambda i: (i, 0, 0)),
            pl.BlockSpec((1, t, dh), lambda i: (i, 0, 0)),
            pl.BlockSpec((1, 2 * dh, buckets + 1), lambda i, h=h: (i % h, 0, 0)),
            pl.BlockSpec((1, 1, dh), lambda i, h=h: (i % h, 0, 0)),
            pl.BlockSpec((1, 1, dh), lambda i, h=h: (i % h, 0, 0)),
        ],
        out_specs=pl.BlockSpec((1, t, dh), lambda i: (i, 0, 0)),
        out_shape=jax.ShapeDtypeStruct((bh, t, dh), jnp.float32),
        compiler_params=pltpu.CompilerParams(
            dimension_semantics=("parallel",)),
    )(qf, kf, vf, w, nk, nv)
    return out.reshape(b, h, t, dh)
